# baseline (speedup 1.0000x reference)
import functools

import jax
import jax.numpy as jnp
import numpy as np
from jax import lax
from jax.experimental import pallas as pl
from jax.experimental.pallas import tpu as pltpu
from jax.experimental.pallas import tpu_sc as plsc

_FM = 32
_RATIOS = [[8, 8], [6, 10], [10, 6], [12, 12], [10, 14], [14, 10], [16, 16], [14, 18], [18, 14]]
_TOTAL = 4041
_PAD = 4048
_P2_COLS = 48
_NEG = float(np.float32(-np.inf))

_GROUPS = [(0, 1867, 3), (1867, 3182, 2), (3182, 4041, 1)]


def _build_consts():
    pk1 = np.zeros(_PAD, np.int32)
    pk2 = np.zeros(_PAD, np.int32)
    rec = np.zeros(_PAD, np.float32)
    p = 0
    for h, w in _RATIOS:
        for i in range(_FM - h + 1):
            for j in range(_FM - w + 1):
                ia = (i + h) * _P2_COLS + (j + w - 1)
                d1 = h * _P2_COLS
                d2 = w - (_P2_COLS if j == 0 else 0)
                pk1[p] = ia | (d1 << 11) | ((d2 + _P2_COLS) << 21)
                pk2[p] = i | (j << 8) | ((i + h - 1) << 16) | ((j + w - 1) << 24)
                rec[p] = np.float32(1.0) / np.float32(h * w)
                p += 1
    assert p == _TOTAL
    pk1[p:] = 47 | (_P2_COLS << 21)
    x0 = pk2 & 0xFF
    y0 = (pk2 >> 8) & 0xFF
    x1 = (pk2 >> 16) & 0xFF
    y1 = (pk2 >> 24) & 0xFF
    war = (x1 - x0 + 1) * (y1 - y0 + 1)
    return np.concatenate([pk1, pk2, war, rec.view(np.int32)])


_CONSTS = _build_consts()

_G0_INT, _G1_INT, _G2_INT = (0, 116), (117, 198), (199, 253)


def _sc_body(x_hbm, c_hbm, ws_hbm, pk_hbm,
             xv, rc, p2, sc, ms, pkv, pkbuf, sem_c):
    wid = lax.axis_index("c") * 16 + lax.axis_index("s")
    cc = pltpu.async_copy(c_hbm, pkv, sem_c)
    pltpu.sync_copy(x_hbm.at[pl.ds(wid * 1024, 1024)], xv)

    iota = lax.broadcasted_iota(jnp.int32, (16,), 0)
    zf = jnp.zeros((16,), jnp.float32)
    zi = jnp.zeros((16,), jnp.int32)
    neg = jnp.full((16,), _NEG, jnp.float32)
    big = jnp.full((16,), 2**30, jnp.int32)

    p2[pl.ds(0, 16)] = zf
    p2[pl.ds(16, 16)] = zf
    p2[pl.ds(32, 16)] = zf

    @plsc.parallel_loop(0, _FM, unroll=2)
    def scan_body(i):
        r0 = xv[pl.ds(i * 32, 16)]
        r1 = xv[pl.ds(i * 32 + 16, 16)]
        rc[pl.ds(i * 32, 16)] = plsc.cumsum(r0)
        rc[pl.ds(i * 32 + 16, 16)] = plsc.cumsum(r1) + jnp.sum(r0)

    def row_body(i, c):
        b = i * _P2_COLS
        p2[pl.ds(b + _P2_COLS, 16)] = p2[pl.ds(b, 16)] + rc[pl.ds(i * 32, 16)]
        p2[pl.ds(b + _P2_COLS + 16, 16)] = p2[pl.ds(b + 16, 16)] + rc[pl.ds(i * 32 + 16, 16)]
        p2[pl.ds(b + _P2_COLS + 32, 16)] = zf
        return c
    lax.fori_loop(0, _FM, row_body, 0, unroll=2)

    cc.wait()

    @plsc.parallel_loop(0, _PAD // 16, unroll=4)
    def pool_body(i):
        o = i * 16
        v = pkv[pl.ds(o, 16)]
        ia = v & 0x7FF
        d1 = (v >> 11) & 0x3FF
        d2 = (v >> 21) - _P2_COLS
        ib = ia - d1
        ic = ia - d2
        idd = ic - d1
        pa = plsc.load_gather(p2, [ia])
        pb = plsc.load_gather(p2, [ib])
        pc = plsc.load_gather(p2, [ic])
        pd = plsc.load_gather(p2, [idd])
        s = (pa - pb - pc + pd) * plsc.bitcast(pkv[pl.ds(3 * _PAD + o, 16)], jnp.float32)
        sc[pl.ds(o, 16)] = s
        ms[pl.ds(o, 16)] = jnp.where(iota + o < _TOTAL, s, neg)

    def track(chunk_i, msv, carry):
        mvec, ivec = carry
        gt = msv > mvec
        return (jnp.where(gt, msv, mvec),
                jnp.where(gt, jnp.full((16,), chunk_i, jnp.int32), ivec))

    def amax_interior(lohi, carry):
        def body(i, carry):
            return track(i, ms[pl.ds(i * 16, 16)], carry)
        return lax.fori_loop(lohi[0], lohi[1], body, carry, unroll=4)

    def unpack_geom(v):
        m8 = jnp.full((16,), 0xFF, jnp.int32)
        gx0 = v & m8
        gy0 = (v >> 8) & m8
        gx1 = (v >> 16) & m8
        gy1 = (v >> 24) & m8
        return gx0, gy0, gx1, gy1

    def sweep_chunk(i, carry, cg, gmask):
        o = i * 16
        msv = ms[pl.ds(o, 16)]
        wx0, wy0, wx1, wy1 = unpack_geom(pkv[pl.ds(_PAD + o, 16)])
        cx0, cy0, cx1, cy1 = cg
        zl = jnp.zeros((16,), jnp.int32)
        lx = jnp.maximum(jnp.minimum(wx1, cx1) - jnp.maximum(wx0, cx0) + 1, zl)
        ly = jnp.maximum(jnp.minimum(wy1, cy1) - jnp.maximum(wy0, cy0) + 1, zl)
        inter = lx * ly
        war = pkv[pl.ds(2 * _PAD + o, 16)]
        carea = (cx1 - cx0 + 1) * (cy1 - cy0 + 1)
        keep = inter * 4 <= war + carea - inter
        if gmask is not None:
            keep = keep | ~gmask
        msv = jnp.where(keep, msv, neg)
        ms[pl.ds(o, 16)] = msv
        if gmask is not None:
            msv = jnp.where(gmask, msv, neg)
        return track(i, msv, carry)

    picks_vec = zi
    slot = 0
    bmask = {116: iota < 11, 198: iota < 14}
    for gi, (lo, hi, npicks) in enumerate(_GROUPS):
        interior = (_G0_INT, _G1_INT, _G2_INT)[gi]
        first_b = 116 if gi == 1 else (198 if gi == 2 else None)
        last_b = 116 if gi == 0 else (198 if gi == 1 else None)

        carry = (neg, zi)
        if first_b is not None:
            mv = jnp.where(~bmask[first_b], ms[pl.ds(first_b * 16, 16)], neg)
            carry = track(first_b, mv, carry)
        carry = amax_interior(interior, carry)
        if last_b is not None:
            mv = jnp.where(bmask[last_b], ms[pl.ds(last_b * 16, 16)], neg)
            carry = track(last_b, mv, carry)
        mvec, ivec = carry

        last = jnp.int32(0)
        for t in range(npicks):
            m = jnp.max(mvec)
            cand = jnp.where(mvec == m, ivec * 16 + iota, big)
            pick = jnp.min(cand)
            if t > 0:
                pick = jnp.where(m > _NEG, pick, last)
            last = pick
            splat = jnp.full((16,), pick, jnp.int32)
            spv = plsc.load_gather(sc, [splat])
            picks_vec = jnp.where(iota == slot, splat, picks_vec)
            picks_vec = jnp.where(iota == 6 + slot, plsc.bitcast(spv, jnp.int32), picks_vec)
            slot += 1
            if t < npicks - 1:
                cg = unpack_geom(plsc.load_gather(pkv, [splat + _PAD]))
                carry = (neg, zi)
                if first_b is not None:
                    carry = sweep_chunk(first_b, carry, cg, ~bmask[first_b])

                @plsc.parallel_loop(interior[0], interior[1], unroll=4, carry=carry)
                def carry(i, c, cg=cg):
                    return sweep_chunk(i, c, cg, None)
                if last_b is not None:
                    carry = sweep_chunk(last_b, carry, cg, bmask[last_b])
                mvec, ivec = carry

    pkbuf[...] = picks_vec
    pltpu.sync_copy(sc.at[pl.ds(0, _TOTAL)], ws_hbm.at[wid])
    pltpu.sync_copy(pkbuf, pk_hbm.at[wid])


@jax.jit
def _launch(x):
    mesh = plsc.VectorSubcoreMesh(core_axis_name="c", subcore_axis_name="s")
    f = functools.partial(
        pl.kernel,
        mesh=mesh,
        compiler_params=pltpu.CompilerParams(
            needs_layout_passes=False, use_tc_tiling_on_sc=False),
        out_type=[
            jax.ShapeDtypeStruct((32, _TOTAL), jnp.float32),
            jax.ShapeDtypeStruct((32, 16), jnp.int32),
        ],
        scratch_types=[
            pltpu.VMEM((1024,), jnp.float32),
            pltpu.VMEM((1024,), jnp.float32),
            pltpu.VMEM((33 * _P2_COLS,), jnp.float32),
            pltpu.VMEM((_PAD,), jnp.float32),
            pltpu.VMEM((_PAD,), jnp.float32),
            pltpu.VMEM((4 * _PAD,), jnp.int32),
            pltpu.VMEM((16,), jnp.int32),
            pltpu.SemaphoreType.DMA,
        ],
    )(_sc_body)
    return f(x, jnp.asarray(_CONSTS))


def kernel(x, proposalN):
    ws, pk = _launch(x.reshape(-1))
    indices = (pk[:, :6] + (proposalN - 6)).astype(jnp.int64)
    pscores = jax.lax.bitcast_convert_type(pk[:, 6:12], jnp.float32)
    return (indices, pscores, ws)


# speedup vs baseline: 1.0075x; 1.0075x over previous
import functools

import jax
import jax.numpy as jnp
import numpy as np
from jax import lax
from jax.experimental import pallas as pl
from jax.experimental.pallas import tpu as pltpu
from jax.experimental.pallas import tpu_sc as plsc

_FM = 32
_RATIOS = [[8, 8], [6, 10], [10, 6], [12, 12], [10, 14], [14, 10], [16, 16], [14, 18], [18, 14]]
_TOTAL = 4041
_PAD = 4048
_P2_COLS = 48
_NEG = float(np.float32(-np.inf))

_GROUPS = [(0, 1867, 3), (1867, 3182, 2), (3182, 4041, 1)]


def _build_consts():
    pk1 = np.zeros(_PAD, np.int32)
    pk2 = np.zeros(_PAD, np.int32)
    rec = np.zeros(_PAD, np.float32)
    p = 0
    for h, w in _RATIOS:
        for i in range(_FM - h + 1):
            for j in range(_FM - w + 1):
                ia = (i + h) * _P2_COLS + (j + w - 1)
                d1 = h * _P2_COLS
                d2 = w - (_P2_COLS if j == 0 else 0)
                pk1[p] = ia | (d1 << 11) | ((d2 + _P2_COLS) << 21)
                pk2[p] = i | (j << 8) | ((i + h - 1) << 16) | ((j + w - 1) << 24)
                rec[p] = np.float32(1.0) / np.float32(h * w)
                p += 1
    assert p == _TOTAL
    pk1[p:] = 47 | (_P2_COLS << 21)
    x0 = pk2 & 0xFF
    y0 = (pk2 >> 8) & 0xFF
    x1 = (pk2 >> 16) & 0xFF
    y1 = (pk2 >> 24) & 0xFF
    war = (x1 - x0 + 1) * (y1 - y0 + 1)
    return np.concatenate([pk1, pk2, war, rec.view(np.int32)])


_CONSTS = _build_consts()

_G0_INT, _G1_INT, _G2_INT = (0, 116), (117, 198), (199, 253)


def _sc_body(x_hbm, c_hbm, ws_hbm, pk_hbm,
             xv, rc, p2, sc, ms, pkv, pkbuf, sem_c):
    wid = lax.axis_index("c") * 16 + lax.axis_index("s")
    cc = pltpu.async_copy(c_hbm, pkv, sem_c)
    pltpu.sync_copy(x_hbm.at[pl.ds(wid * 1024, 1024)], xv)

    iota = lax.broadcasted_iota(jnp.int32, (16,), 0)
    zf = jnp.zeros((16,), jnp.float32)
    zi = jnp.zeros((16,), jnp.int32)
    neg = jnp.full((16,), _NEG, jnp.float32)
    big = jnp.full((16,), 2**30, jnp.int32)

    p2[pl.ds(0, 16)] = zf
    p2[pl.ds(16, 16)] = zf
    p2[pl.ds(32, 16)] = zf

    @plsc.parallel_loop(0, _FM, unroll=2)
    def scan_body(i):
        r0 = xv[pl.ds(i * 32, 16)]
        r1 = xv[pl.ds(i * 32 + 16, 16)]
        rc[pl.ds(i * 32, 16)] = plsc.cumsum(r0)
        rc[pl.ds(i * 32 + 16, 16)] = plsc.cumsum(r1) + jnp.sum(r0)

    def row_body(i, c):
        b = i * _P2_COLS
        p2[pl.ds(b + _P2_COLS, 16)] = p2[pl.ds(b, 16)] + rc[pl.ds(i * 32, 16)]
        p2[pl.ds(b + _P2_COLS + 16, 16)] = p2[pl.ds(b + 16, 16)] + rc[pl.ds(i * 32 + 16, 16)]
        p2[pl.ds(b + _P2_COLS + 32, 16)] = zf
        return c
    lax.fori_loop(0, _FM, row_body, 0, unroll=2)

    cc.wait()

    @plsc.parallel_loop(0, _PAD // 16, unroll=4)
    def pool_body(i):
        o = i * 16
        v = pkv[pl.ds(o, 16)]
        ia = v & 0x7FF
        d1 = (v >> 11) & 0x3FF
        d2 = (v >> 21) - _P2_COLS
        ib = ia - d1
        ic = ia - d2
        idd = ic - d1
        pa = plsc.load_gather(p2, [ia])
        pb = plsc.load_gather(p2, [ib])
        pc = plsc.load_gather(p2, [ic])
        pd = plsc.load_gather(p2, [idd])
        s = (pa - pb - pc + pd) * plsc.bitcast(pkv[pl.ds(3 * _PAD + o, 16)], jnp.float32)
        sc[pl.ds(o, 16)] = s
        ms[pl.ds(o, 16)] = jnp.where(iota + o < _TOTAL, s, neg)

    def track(chunk_i, msv, carry):
        mvec, ivec = carry
        gt = msv > mvec
        return (jnp.where(gt, msv, mvec),
                jnp.where(gt, jnp.full((16,), chunk_i, jnp.int32), ivec))

    def amax_interior(lohi, carry):
        def body(i, carry):
            return track(i, ms[pl.ds(i * 16, 16)], carry)
        return lax.fori_loop(lohi[0], lohi[1], body, carry, unroll=2)

    def unpack_geom(v):
        m8 = jnp.full((16,), 0xFF, jnp.int32)
        gx0 = v & m8
        gy0 = (v >> 8) & m8
        gx1 = (v >> 16) & m8
        gy1 = (v >> 24) & m8
        return gx0, gy0, gx1, gy1

    def sweep_chunk(i, carry, cg, gmask):
        o = i * 16
        msv = ms[pl.ds(o, 16)]
        wx0, wy0, wx1, wy1 = unpack_geom(pkv[pl.ds(_PAD + o, 16)])
        cx0, cy0, cx1, cy1 = cg
        zl = jnp.zeros((16,), jnp.int32)
        lx = jnp.maximum(jnp.minimum(wx1, cx1) - jnp.maximum(wx0, cx0) + 1, zl)
        ly = jnp.maximum(jnp.minimum(wy1, cy1) - jnp.maximum(wy0, cy0) + 1, zl)
        inter = lx * ly
        war = pkv[pl.ds(2 * _PAD + o, 16)]
        carea = (cx1 - cx0 + 1) * (cy1 - cy0 + 1)
        keep = inter * 4 <= war + carea - inter
        if gmask is not None:
            keep = keep | ~gmask
        msv = jnp.where(keep, msv, neg)
        ms[pl.ds(o, 16)] = msv
        if gmask is not None:
            msv = jnp.where(gmask, msv, neg)
        return track(i, msv, carry)

    picks_vec = zi
    slot = 0
    bmask = {116: iota < 11, 198: iota < 14}
    for gi, (lo, hi, npicks) in enumerate(_GROUPS):
        interior = (_G0_INT, _G1_INT, _G2_INT)[gi]
        first_b = 116 if gi == 1 else (198 if gi == 2 else None)
        last_b = 116 if gi == 0 else (198 if gi == 1 else None)

        carry = (neg, zi)
        if first_b is not None:
            mv = jnp.where(~bmask[first_b], ms[pl.ds(first_b * 16, 16)], neg)
            carry = track(first_b, mv, carry)
        carry = amax_interior(interior, carry)
        if last_b is not None:
            mv = jnp.where(bmask[last_b], ms[pl.ds(last_b * 16, 16)], neg)
            carry = track(last_b, mv, carry)
        mvec, ivec = carry

        last = jnp.int32(0)
        for t in range(npicks):
            m = jnp.max(mvec)
            cand = jnp.where(mvec == m, ivec * 16 + iota, big)
            pick = jnp.min(cand)
            if t > 0:
                pick = jnp.where(m > _NEG, pick, last)
            last = pick
            splat = jnp.full((16,), pick, jnp.int32)
            spv = plsc.load_gather(sc, [splat])
            picks_vec = jnp.where(iota == slot, splat, picks_vec)
            picks_vec = jnp.where(iota == 6 + slot, plsc.bitcast(spv, jnp.int32), picks_vec)
            slot += 1
            if t < npicks - 1:
                cg = unpack_geom(plsc.load_gather(pkv, [splat + _PAD]))
                carry = (neg, zi)
                if first_b is not None:
                    carry = sweep_chunk(first_b, carry, cg, ~bmask[first_b])

                @plsc.parallel_loop(interior[0], interior[1], unroll=2, carry=carry)
                def carry(i, c, cg=cg):
                    return sweep_chunk(i, c, cg, None)
                if last_b is not None:
                    carry = sweep_chunk(last_b, carry, cg, bmask[last_b])
                mvec, ivec = carry

    pkbuf[...] = picks_vec
    pltpu.sync_copy(sc.at[pl.ds(0, _TOTAL)], ws_hbm.at[wid])
    pltpu.sync_copy(pkbuf, pk_hbm.at[wid])


@jax.jit
def _launch(x):
    mesh = plsc.VectorSubcoreMesh(core_axis_name="c", subcore_axis_name="s")
    f = functools.partial(
        pl.kernel,
        mesh=mesh,
        compiler_params=pltpu.CompilerParams(
            needs_layout_passes=False, use_tc_tiling_on_sc=False),
        out_type=[
            jax.ShapeDtypeStruct((32, _TOTAL), jnp.float32),
            jax.ShapeDtypeStruct((32, 16), jnp.int32),
        ],
        scratch_types=[
            pltpu.VMEM((1024,), jnp.float32),
            pltpu.VMEM((1024,), jnp.float32),
            pltpu.VMEM((33 * _P2_COLS,), jnp.float32),
            pltpu.VMEM((_PAD,), jnp.float32),
            pltpu.VMEM((_PAD,), jnp.float32),
            pltpu.VMEM((4 * _PAD,), jnp.int32),
            pltpu.VMEM((16,), jnp.int32),
            pltpu.SemaphoreType.DMA,
        ],
    )(_sc_body)
    return f(x, jnp.asarray(_CONSTS))


def kernel(x, proposalN):
    ws, pk = _launch(x.reshape(-1))
    indices = (pk[:, :6] + (proposalN - 6)).astype(jnp.int64)
    pscores = jax.lax.bitcast_convert_type(pk[:, 6:12], jnp.float32)
    return (indices, pscores, ws)
